# baseline (speedup 1.0000x reference)
import functools

import jax
import jax.numpy as jnp
from jax import lax
from jax.experimental import pallas as pl
from jax.experimental.pallas import tpu as pltpu
from jax.experimental.pallas import tpu_sc as plsc

N = 2048
C = 1024
I = 512
E = 8
K = 2
BM = 256
GBUF = N * K + E * BM
NBLK = GBUF // BM
NW = 32
PAIRS = N * K


def _cumsum_rows(a, n):
    s = 1
    while s < n:
        shifted = jnp.concatenate(
            [jnp.zeros((s, a.shape[1]), a.dtype), a[:-s, :]], axis=0)
        a = a + shifted
        s *= 2
    return a


def _lane_cumsum(v):
    for s in (1, 2, 4):
        v = v + jnp.concatenate(
            [jnp.zeros((1, s), v.dtype), v[:, :-s]], axis=1)
    return v


def _router_body(x_ref, wg_ref, p0_ref, p1_ref, pos0_ref, pos1_ref, be_ref):
    x = x_ref[...]
    wg = wg_ref[...]
    logits = lax.dot_general(
        x, wg, (((1,), (1,)), ((), ())), preferred_element_type=jnp.float32
    )
    n, e = logits.shape
    eidx = lax.broadcasted_iota(jnp.int32, (n, e), 1)
    m0 = jnp.max(logits, axis=1, keepdims=True)
    i0 = jnp.min(jnp.where(logits == m0, eidx, e), axis=1, keepdims=True)
    masked = jnp.where(eidx == i0, -jnp.inf, logits)
    m1 = jnp.max(masked, axis=1, keepdims=True)
    i1 = jnp.min(jnp.where(masked == m1, eidx, e), axis=1, keepdims=True)
    t = jnp.exp(m1 - m0)
    p0 = 1.0 / (1.0 + t)
    p0_ref[...] = p0
    p1_ref[...] = t * p0

    oh0 = (eidx == i0).astype(jnp.float32)
    oh1 = (eidx == i1).astype(jnp.float32)
    c0 = _cumsum_rows(oh0, n)
    c1 = _cumsum_rows(oh1, n)
    s0 = c0 - oh0
    tot0 = c0[n - 1:n, :]
    s1 = tot0 + (c1 - oh1)
    tot = tot0 + c1[n - 1:n, :]
    padded = jnp.ceil(tot * (1.0 / BM)) * BM
    inc = _lane_cumsum(padded)
    offs = inc - padded
    pos0 = jnp.sum(oh0 * (s0 + offs), axis=1, keepdims=True)
    pos1 = jnp.sum(oh1 * (s1 + offs), axis=1, keepdims=True)
    pos0_ref[...] = pos0.astype(jnp.int32)
    pos1_ref[...] = pos1.astype(jnp.int32)

    inct = jnp.transpose(inc)
    bpos = lax.broadcasted_iota(jnp.int32, (e, 128), 1).astype(jnp.float32) * BM
    be = jnp.sum((bpos >= inct).astype(jnp.int32), axis=0, keepdims=True)
    be_ref[...] = be


def _mlp(xb, w1, w2, w3):
    h1 = lax.dot(xb, w1, preferred_element_type=jnp.float32)
    h2 = lax.dot(xb, w2, preferred_element_type=jnp.float32)
    g = (h1 * (1.0 / (1.0 + jnp.exp(-h1)))).astype(jnp.bfloat16)
    return lax.dot(g * h2.astype(jnp.bfloat16), w3,
                   preferred_element_type=jnp.float32)


def _gmm_body(be_ref, xs_ref, w1_ref, w2_ref, w3_ref, out_ref,
              w1b, w2b, w3b):
    i = pl.program_id(0)
    be = be_ref[i]
    prev = be_ref[jnp.maximum(i - 1, 0)]
    changed = jnp.logical_or(i == 0, be != prev)

    @pl.when(jnp.logical_and(changed, be < E))
    def _():
        w1b[...] = w1_ref[0].astype(jnp.bfloat16)
        w2b[...] = w2_ref[0].astype(jnp.bfloat16)
        w3b[...] = w3_ref[0].astype(jnp.bfloat16)

    @pl.when(be < E)
    def _():
        xb = xs_ref[...].astype(jnp.bfloat16)
        out_ref[...] = _mlp(xb, w1b[...], w2b[...], w3b[...])


def _shared_body(xb_ref, ws1_ref, ws2_ref, ws3_ref, s_ref):
    s_ref[...] = _mlp(xb_ref[...],
                      ws1_ref[...].astype(jnp.bfloat16),
                      ws2_ref[...].astype(jnp.bfloat16),
                      ws3_ref[...].astype(jnp.bfloat16))


def _combine_body(r0_ref, r1_ref, s_ref, p0_ref, p1_ref, y_ref):
    y_ref[...] = (p0_ref[...] * r0_ref[...] + p1_ref[...] * r1_ref[...]
                  + s_ref[...])


def _sc_mesh():
    return plsc.VectorSubcoreMesh(core_axis_name="c", subcore_axis_name="s")


def _dispatch_call(x_flat, pos_pairs):
    tpw = N * K // NW
    nch = 4
    cw = tpw // nch

    @functools.partial(
        pl.kernel,
        out_type=jax.ShapeDtypeStruct((GBUF, C), jnp.float32),
        mesh=_sc_mesh(),
        scratch_types=[
            pltpu.VMEM((nch, cw), jnp.int32),
            pltpu.VMEM((cw, C), jnp.float32),
            pltpu.VMEM((cw, C), jnp.float32),
            pltpu.SemaphoreType.DMA,
            pltpu.SemaphoreType.DMA,
        ],
    )
    def dispatch(x_hbm, pos_hbm, xs_hbm, idx_v, buf0, buf1, sem0, sem1):
        w = lax.axis_index("s") * 2 + lax.axis_index("c")
        t0 = (w % (N // tpw)) * tpw
        pltpu.sync_copy(pos_hbm.at[w], idx_v)
        bufs = (buf0, buf1)
        sems = (sem0, sem1)
        cps = [None, None]
        for cc in range(nch):
            b = cc % 2
            if cps[b] is not None:
                cps[b].wait()
            pltpu.sync_copy(x_hbm.at[pl.ds(t0 + cc * cw, cw)], bufs[b])
            cps[b] = pltpu.async_copy(bufs[b], xs_hbm.at[idx_v.at[cc]],
                                      sems[b])
        for cp in cps:
            cp.wait()

    return dispatch(x_flat, pos_pairs)


def _gather_call(rows, pos0_w, pos1_w):
    tpw = N // NW
    nch = 4
    cw = tpw // nch

    @functools.partial(
        pl.kernel,
        out_type=(jax.ShapeDtypeStruct((N, C), jnp.float32),
                  jax.ShapeDtypeStruct((N, C), jnp.float32)),
        mesh=_sc_mesh(),
        scratch_types=[
            pltpu.VMEM((nch, cw), jnp.int32),
            pltpu.VMEM((nch, cw), jnp.int32),
            pltpu.VMEM((cw, C), jnp.float32),
            pltpu.VMEM((cw, C), jnp.float32),
            pltpu.SemaphoreType.DMA,
            pltpu.SemaphoreType.DMA,
        ],
    )
    def gather2(rows_hbm, pos0_hbm, pos1_hbm, r0_hbm, r1_hbm,
                i0v, i1v, b0, b1, s0, s1):
        w = lax.axis_index("s") * 2 + lax.axis_index("c")
        t0 = w * tpw
        pltpu.sync_copy(pos0_hbm.at[w], i0v)
        pltpu.sync_copy(pos1_hbm.at[w], i1v)
        for cc in range(nch):
            cp0 = pltpu.async_copy(rows_hbm.at[i0v.at[cc]], b0, s0)
            cp1 = pltpu.async_copy(rows_hbm.at[i1v.at[cc]], b1, s1)
            cp0.wait()
            cp1.wait()
            pltpu.sync_copy(b0, r0_hbm.at[pl.ds(t0 + cc * cw, cw)])
            pltpu.sync_copy(b1, r1_hbm.at[pl.ds(t0 + cc * cw, cw)])

    return gather2(rows, pos0_w, pos1_w)


def kernel(x, Wg, W1, W2, W3, Ws1, Ws2, Ws3):
    Bb, Tt, Cc = x.shape
    x_flat = x.reshape(N, C)
    x_bf = x_flat.astype(jnp.bfloat16)

    p0, p1, pos0, pos1, be = pl.pallas_call(
        _router_body,
        out_shape=(
            jax.ShapeDtypeStruct((N, 1), jnp.float32),
            jax.ShapeDtypeStruct((N, 1), jnp.float32),
            jax.ShapeDtypeStruct((N, 1), jnp.int32),
            jax.ShapeDtypeStruct((N, 1), jnp.int32),
            jax.ShapeDtypeStruct((1, 128), jnp.int32),
        ),
    )(x_flat, Wg)

    be_vec = be[0, :NBLK]
    pos_pairs = jnp.concatenate([pos0, pos1], axis=0).reshape(NW, 4, -1)
    pos0_w = pos0.reshape(NW, 4, -1)
    pos1_w = pos1.reshape(NW, 4, -1)

    xs = _dispatch_call(x_flat, pos_pairs)

    rows = pl.pallas_call(
        _gmm_body,
        grid_spec=pltpu.PrefetchScalarGridSpec(
            num_scalar_prefetch=1,
            grid=(NBLK,),
            in_specs=[
                pl.BlockSpec((BM, C), lambda i, be_s: (i, 0)),
                pl.BlockSpec((1, C, I),
                             lambda i, be_s: (jnp.minimum(be_s[i], E - 1), 0, 0)),
                pl.BlockSpec((1, C, I),
                             lambda i, be_s: (jnp.minimum(be_s[i], E - 1), 0, 0)),
                pl.BlockSpec((1, I, C),
                             lambda i, be_s: (jnp.minimum(be_s[i], E - 1), 0, 0)),
            ],
            out_specs=pl.BlockSpec((BM, C), lambda i, be_s: (i, 0)),
            scratch_shapes=[
                pltpu.VMEM((C, I), jnp.bfloat16),
                pltpu.VMEM((C, I), jnp.bfloat16),
                pltpu.VMEM((I, C), jnp.bfloat16),
            ],
        ),
        out_shape=jax.ShapeDtypeStruct((GBUF, C), jnp.float32),
    )(be_vec, xs, W1, W2, W3)

    BN = 512
    s_out = pl.pallas_call(
        _shared_body,
        grid=(N // BN,),
        in_specs=[
            pl.BlockSpec((BN, C), lambda i: (i, 0)),
            pl.BlockSpec((C, I), lambda i: (0, 0)),
            pl.BlockSpec((C, I), lambda i: (0, 0)),
            pl.BlockSpec((I, C), lambda i: (0, 0)),
        ],
        out_specs=pl.BlockSpec((BN, C), lambda i: (i, 0)),
        out_shape=jax.ShapeDtypeStruct((N, C), jnp.float32),
    )(x_bf, Ws1, Ws2, Ws3)

    r0, r1 = _gather_call(rows, pos0_w, pos1_w)

    y = pl.pallas_call(
        _combine_body,
        grid=(N // BN,),
        in_specs=[
            pl.BlockSpec((BN, C), lambda i: (i, 0)),
            pl.BlockSpec((BN, C), lambda i: (i, 0)),
            pl.BlockSpec((BN, C), lambda i: (i, 0)),
            pl.BlockSpec((BN, 1), lambda i: (i, 0)),
            pl.BlockSpec((BN, 1), lambda i: (i, 0)),
        ],
        out_specs=pl.BlockSpec((BN, C), lambda i: (i, 0)),
        out_shape=jax.ShapeDtypeStruct((N, C), jnp.float32),
    )(r0, r1, s_out, p0, p1)

    return y.reshape(Bb, Tt, Cc)


# speedup vs baseline: 1.0147x; 1.0147x over previous
import functools

import jax
import jax.numpy as jnp
from jax import lax
from jax.experimental import pallas as pl
from jax.experimental.pallas import tpu as pltpu
from jax.experimental.pallas import tpu_sc as plsc

N = 2048
C = 1024
I = 512
E = 8
K = 2
BM = 256
GBUF = N * K + E * BM
NBLK = GBUF // BM
NW = 32
PAIRS = N * K
CP = C // 2


def _cumsum_rows(a, n):
    s = 1
    while s < n:
        shifted = jnp.concatenate(
            [jnp.zeros((s, a.shape[1]), a.dtype), a[:-s, :]], axis=0)
        a = a + shifted
        s *= 2
    return a


def _lane_cumsum(v):
    for s in (1, 2, 4):
        v = v + jnp.concatenate(
            [jnp.zeros((1, s), v.dtype), v[:, :-s]], axis=1)
    return v


def _router_body(x_ref, wg_ref, p0_ref, p1_ref, pos0_ref, pos1_ref, be_ref):
    x = x_ref[...]
    wg = wg_ref[...]
    logits = lax.dot_general(
        x, wg, (((1,), (1,)), ((), ())), preferred_element_type=jnp.float32
    )
    n, e = logits.shape
    eidx = lax.broadcasted_iota(jnp.int32, (n, e), 1)
    m0 = jnp.max(logits, axis=1, keepdims=True)
    i0 = jnp.min(jnp.where(logits == m0, eidx, e), axis=1, keepdims=True)
    masked = jnp.where(eidx == i0, -jnp.inf, logits)
    m1 = jnp.max(masked, axis=1, keepdims=True)
    i1 = jnp.min(jnp.where(masked == m1, eidx, e), axis=1, keepdims=True)
    t = jnp.exp(m1 - m0)
    p0 = 1.0 / (1.0 + t)
    p0_ref[...] = p0
    p1_ref[...] = t * p0

    oh0 = (eidx == i0).astype(jnp.float32)
    oh1 = (eidx == i1).astype(jnp.float32)
    c0 = _cumsum_rows(oh0, n)
    c1 = _cumsum_rows(oh1, n)
    s0 = c0 - oh0
    tot0 = c0[n - 1:n, :]
    s1 = tot0 + (c1 - oh1)
    tot = tot0 + c1[n - 1:n, :]
    padded = jnp.ceil(tot * (1.0 / BM)) * BM
    inc = _lane_cumsum(padded)
    offs = inc - padded
    pos0 = jnp.sum(oh0 * (s0 + offs), axis=1, keepdims=True)
    pos1 = jnp.sum(oh1 * (s1 + offs), axis=1, keepdims=True)
    pos0_ref[...] = pos0.astype(jnp.int32)
    pos1_ref[...] = pos1.astype(jnp.int32)

    inct = jnp.transpose(inc)
    bpos = lax.broadcasted_iota(jnp.int32, (e, 128), 1).astype(jnp.float32) * BM
    be = jnp.sum((bpos >= inct).astype(jnp.int32), axis=0, keepdims=True)
    be_ref[...] = be


def _mlp(xb, w1, w2, w3):
    h1 = lax.dot(xb, w1, preferred_element_type=jnp.float32)
    h2 = lax.dot(xb, w2, preferred_element_type=jnp.float32)
    g = (h1 * (1.0 / (1.0 + jnp.exp(-h1)))).astype(jnp.bfloat16)
    return lax.dot(g * h2.astype(jnp.bfloat16), w3,
                   preferred_element_type=jnp.float32)


def _gmm_body(be_ref, xs_ref, w1_ref, w2_ref, w3_ref, out_ref,
              w1b, w2b, w3b):
    i = pl.program_id(0)
    be = be_ref[i]
    prev = be_ref[jnp.maximum(i - 1, 0)]
    changed = jnp.logical_or(i == 0, be != prev)

    @pl.when(jnp.logical_and(changed, be < E))
    def _():
        w1b[...] = w1_ref[0].astype(jnp.bfloat16)
        w2b[...] = w2_ref[0].astype(jnp.bfloat16)
        w3b[...] = w3_ref[0].astype(jnp.bfloat16)

    @pl.when(be < E)
    def _():
        xb = xs_ref[...].astype(jnp.bfloat16)
        out_ref[...] = _mlp(xb, w1b[...], w2b[...], w3b[...])


def _combine_body(r0_ref, r1_ref, xb_ref, ws1_ref, ws2_ref, ws3_ref,
                  p0_ref, p1_ref, y_ref):
    s = _mlp(xb_ref[...],
             ws1_ref[...].astype(jnp.bfloat16),
             ws2_ref[...].astype(jnp.bfloat16),
             ws3_ref[...].astype(jnp.bfloat16))
    y_ref[...] = (p0_ref[...] * r0_ref[...]
                  + p1_ref[...] * r1_ref[...] + s)


def _sc_mesh():
    return plsc.VectorSubcoreMesh(core_axis_name="c", subcore_axis_name="s")


def _dispatch_call(x_bf, pos_pairs):
    tpw = PAIRS // NW

    nch = 4
    cw = tpw // nch

    @functools.partial(
        pl.kernel,
        out_type=jax.ShapeDtypeStruct((GBUF, C), jnp.float32),
        mesh=_sc_mesh(),
        scratch_types=[
            pltpu.VMEM((nch, cw), jnp.int32),
            pltpu.VMEM((cw, C), jnp.float32),
            pltpu.VMEM((cw, C), jnp.float32),
            pltpu.SemaphoreType.DMA,
            pltpu.SemaphoreType.DMA,
        ],
    )
    def dispatch(x_hbm, pos_hbm, xs_hbm, idx_v, buf0, buf1, sem0, sem1):
        w = lax.axis_index("s") * 2 + lax.axis_index("c")
        t0 = (w % (N // tpw)) * tpw
        pltpu.sync_copy(pos_hbm.at[w], idx_v)
        bufs = (buf0, buf1)
        sems = (sem0, sem1)
        cps = [None, None]
        for cc in range(nch):
            b = cc % 2
            if cps[b] is not None:
                cps[b].wait()
            pltpu.sync_copy(x_hbm.at[pl.ds(t0 + cc * cw, cw)], bufs[b])
            cps[b] = pltpu.async_copy(bufs[b], xs_hbm.at[idx_v.at[cc]],
                                      sems[b])
        for cp in cps:
            cp.wait()

    return dispatch(x_bf, pos_pairs)


def _gather_call(rows, pos0_w, pos1_w):
    tpw = N // NW

    nch = 2
    cw = tpw // nch

    @functools.partial(
        pl.kernel,
        out_type=(jax.ShapeDtypeStruct((N, C), jnp.float32),
                  jax.ShapeDtypeStruct((N, C), jnp.float32)),
        mesh=_sc_mesh(),
        scratch_types=[
            pltpu.VMEM((nch, cw), jnp.int32),
            pltpu.VMEM((nch, cw), jnp.int32),
            pltpu.VMEM((cw, C), jnp.float32),
            pltpu.VMEM((cw, C), jnp.float32),
            pltpu.SemaphoreType.DMA,
            pltpu.SemaphoreType.DMA,
        ],
    )
    def gather2(rows_hbm, pos0_hbm, pos1_hbm, r0_hbm, r1_hbm,
                i0v, i1v, b0, b1, s0, s1):
        w = lax.axis_index("s") * 2 + lax.axis_index("c")
        t0 = w * tpw
        pltpu.sync_copy(pos0_hbm.at[w], i0v)
        pltpu.sync_copy(pos1_hbm.at[w], i1v)
        for cc in range(nch):
            cp0 = pltpu.async_copy(rows_hbm.at[i0v.at[cc]], b0, s0)
            cp1 = pltpu.async_copy(rows_hbm.at[i1v.at[cc]], b1, s1)
            cp0.wait()
            cp1.wait()
            pltpu.sync_copy(b0, r0_hbm.at[pl.ds(t0 + cc * cw, cw)])
            pltpu.sync_copy(b1, r1_hbm.at[pl.ds(t0 + cc * cw, cw)])

    return gather2(rows, pos0_w, pos1_w)


def kernel(x, Wg, W1, W2, W3, Ws1, Ws2, Ws3):
    Bb, Tt, Cc = x.shape
    x_flat = x.reshape(N, C)
    x_bf = x_flat.astype(jnp.bfloat16)

    p0, p1, pos0, pos1, be = pl.pallas_call(
        _router_body,
        out_shape=(
            jax.ShapeDtypeStruct((N, 1), jnp.float32),
            jax.ShapeDtypeStruct((N, 1), jnp.float32),
            jax.ShapeDtypeStruct((N, 1), jnp.int32),
            jax.ShapeDtypeStruct((N, 1), jnp.int32),
            jax.ShapeDtypeStruct((1, 128), jnp.int32),
        ),
    )(x_flat, Wg)

    be_vec = be[0, :NBLK]
    pos_pairs = jnp.concatenate([pos0, pos1], axis=0).reshape(NW, 4, -1)
    pos0_w = pos0.reshape(NW, 2, -1)
    pos1_w = pos1.reshape(NW, 2, -1)

    xs = _dispatch_call(x_flat, pos_pairs)

    rows = pl.pallas_call(
        _gmm_body,
        grid_spec=pltpu.PrefetchScalarGridSpec(
            num_scalar_prefetch=1,
            grid=(NBLK,),
            in_specs=[
                pl.BlockSpec((BM, C), lambda i, be_s: (i, 0)),
                pl.BlockSpec((1, C, I),
                             lambda i, be_s: (jnp.minimum(be_s[i], E - 1), 0, 0)),
                pl.BlockSpec((1, C, I),
                             lambda i, be_s: (jnp.minimum(be_s[i], E - 1), 0, 0)),
                pl.BlockSpec((1, I, C),
                             lambda i, be_s: (jnp.minimum(be_s[i], E - 1), 0, 0)),
            ],
            out_specs=pl.BlockSpec((BM, C), lambda i, be_s: (i, 0)),
            scratch_shapes=[
                pltpu.VMEM((C, I), jnp.bfloat16),
                pltpu.VMEM((C, I), jnp.bfloat16),
                pltpu.VMEM((I, C), jnp.bfloat16),
            ],
        ),
        out_shape=jax.ShapeDtypeStruct((GBUF, C), jnp.float32),
    )(be_vec, xs, W1, W2, W3)

    r0, r1 = _gather_call(rows, pos0_w, pos1_w)

    BN = 512
    y = pl.pallas_call(
        _combine_body,
        grid=(N // BN,),
        in_specs=[
            pl.BlockSpec((BN, C), lambda i: (i, 0)),
            pl.BlockSpec((BN, C), lambda i: (i, 0)),
            pl.BlockSpec((BN, C), lambda i: (i, 0)),
            pl.BlockSpec((C, I), lambda i: (0, 0)),
            pl.BlockSpec((C, I), lambda i: (0, 0)),
            pl.BlockSpec((I, C), lambda i: (0, 0)),
            pl.BlockSpec((BN, 1), lambda i: (i, 0)),
            pl.BlockSpec((BN, 1), lambda i: (i, 0)),
        ],
        out_specs=pl.BlockSpec((BN, C), lambda i: (i, 0)),
        out_shape=jax.ShapeDtypeStruct((N, C), jnp.float32),
    )(r0, r1, x_bf, Ws1, Ws2, Ws3, p0, p1)

    return y.reshape(Bb, Tt, Cc)
